# Initial kernel scaffold; baseline (speedup 1.0000x reference)
import jax
import jax.numpy as jnp
from jax.experimental import pallas as pl


def kernel(efeat, nfeat, edge_index, W1, b1, W2, b2, ln_g, ln_b):
    raise NotImplementedError("write your pallas kernel here")


# speedup vs baseline: 1.8038x; 1.8038x over previous
import functools

import jax
import jax.numpy as jnp
from jax import lax
from jax.experimental import pallas as pl
from jax.experimental.pallas import tpu as pltpu
from jax.experimental.pallas import tpu_sc as plsc

_NC = 2
_NS = 16
_NW = _NC * _NS
_CH = 128


def _node_proj(nfeat, w1_src, w1_dst):
    n, d = nfeat.shape
    h = w1_src.shape[1]

    def body(n_ref, ws_ref, wd_ref, t_ref):
        x = n_ref[...]
        t_ref[0:n, :] = jnp.dot(x, ws_ref[...], preferred_element_type=jnp.float32)
        t_ref[n:2 * n, :] = jnp.dot(x, wd_ref[...], preferred_element_type=jnp.float32)

    return pl.pallas_call(
        body,
        out_shape=jax.ShapeDtypeStruct((2 * n, h), jnp.float32),
    )(nfeat, w1_src, w1_dst)


def _sc_gather(table, idx):
    total = idx.shape[0]
    d = table.shape[1]
    chunks_per_w = total // (_NW * _CH)
    mesh = plsc.VectorSubcoreMesh(core_axis_name="c", subcore_axis_name="s")

    @functools.partial(
        pl.kernel,
        mesh=mesh,
        out_type=jax.ShapeDtypeStruct((total, d), jnp.float32),
        scratch_types=[
            pltpu.VMEM((_CH,), jnp.int32),
            pltpu.VMEM((_CH, d), jnp.float32),
            pltpu.SemaphoreType.DMA,
        ],
    )
    def gather_k(t_hbm, idx_hbm, out_hbm, idx_v, rows_v, sem):
        wid = lax.axis_index("s") * _NC + lax.axis_index("c")
        base = wid * chunks_per_w * _CH

        @pl.loop(0, chunks_per_w)
        def _(i):
            off = base + i * _CH
            pltpu.sync_copy(idx_hbm.at[pl.ds(off, _CH)], idx_v)
            pltpu.async_copy(t_hbm.at[idx_v], rows_v, sem).wait()
            pltpu.sync_copy(rows_v, out_hbm.at[pl.ds(off, _CH)])

    return gather_k(table, idx)


def _edge_mlp(efeat, rows, w1_edge, w2, b1, b2, ln_g, ln_b, block):
    e, d = efeat.shape
    dst_block_off = e // block

    def body(x_ref, rs_ref, rd_ref, we_ref, w2_ref, b1_ref, b2_ref,
             g_ref, bb_ref, o_ref):
        x = x_ref[...]
        h = jnp.dot(x, we_ref[...], preferred_element_type=jnp.float32)
        h = h + rs_ref[...] + rd_ref[...] + b1_ref[...]
        h = h * jax.nn.sigmoid(h)
        h2 = jnp.dot(h, w2_ref[...], preferred_element_type=jnp.float32)
        h2 = h2 + b2_ref[...]
        mu = jnp.mean(h2, axis=-1, keepdims=True)
        var = jnp.mean((h2 - mu) * (h2 - mu), axis=-1, keepdims=True)
        o_ref[...] = (h2 - mu) * lax.rsqrt(var + 1e-5) * g_ref[...] + bb_ref[...] + x

    hid = w1_edge.shape[1]
    out_dim = w2.shape[1]
    full = lambda *s: pl.BlockSpec(s, lambda i: tuple(0 for _ in s))
    return pl.pallas_call(
        body,
        grid=(e // block,),
        in_specs=[
            pl.BlockSpec((block, d), lambda i: (i, 0)),
            pl.BlockSpec((block, hid), lambda i: (i, 0)),
            pl.BlockSpec((block, hid), lambda i: (i + dst_block_off, 0)),
            full(d, hid),
            full(hid, out_dim),
            full(1, hid),
            full(1, out_dim),
            full(1, out_dim),
            full(1, out_dim),
        ],
        out_specs=pl.BlockSpec((block, out_dim), lambda i: (i, 0)),
        out_shape=jax.ShapeDtypeStruct((e, out_dim), jnp.float32),
        compiler_params=pltpu.CompilerParams(
            dimension_semantics=("parallel",),
        ),
    )(efeat, rows, rows, w1_edge, w2, b1, b2, ln_g, ln_b)


def kernel(efeat, nfeat, edge_index, W1, b1, W2, b2, ln_g, ln_b):
    e, d_edge = efeat.shape
    n, d_node = nfeat.shape
    src = edge_index[0]
    dst = edge_index[1]

    table = _node_proj(nfeat, W1[d_edge:d_edge + d_node], W1[d_edge + d_node:])

    quantum = _NW * _CH
    total = ((2 * e + quantum - 1) // quantum) * quantum
    pad = total - 2 * e
    idx = jnp.concatenate(
        [src, dst + n, jnp.zeros((pad,), dtype=jnp.int32)])
    rows = _sc_gather(table, idx)

    out = _edge_mlp(
        efeat, rows, W1[:d_edge], W2,
        b1.reshape(1, -1), b2.reshape(1, -1),
        ln_g.reshape(1, -1), ln_b.reshape(1, -1), block=512)
    return (out, nfeat)
